# baseline (device time: 178187 ns/iter reference)
import jax
import jax.numpy as jnp
from jax import lax
from jax.experimental import pallas as pl
from jax.experimental.pallas import tpu as pltpu

N_DEV = 16
SQ = 1024
SKV = 1024
HQ_LOCAL = 8
DH = 128
D_MODEL = 1024
CHUNK = SQ // N_DEV
SCALE = 0.08838834764831843


def kernel(x, Wq, K_ext, V_ext, Wo):
    i = lax.axis_index("i")

    x2 = x[0]
    wq_sl = lax.dynamic_slice(Wq, (0, i * D_MODEL), (D_MODEL, D_MODEL))
    wo_sl = lax.dynamic_slice(Wo, (i * D_MODEL, 0), (D_MODEL, D_MODEL))
    k2 = K_ext[0].reshape(SKV, HQ_LOCAL * DH)
    v2 = V_ext[0].reshape(SKV, HQ_LOCAL * DH)

    def body(x_ref, wq_ref, k_ref, v_ref, wo_ref, out_ref,
             acc_ref, ctx_ref, comm_ref,
             rs_send, rs_recv, ag_send, ag_recv):
        my = lax.axis_index("i")
        left = lax.rem(my + N_DEV - 1, N_DEV)
        right = lax.rem(my + 1, N_DEV)

        barrier_sem = pltpu.get_barrier_semaphore()
        for nbr in (left, right):
            pl.semaphore_signal(
                barrier_sem, inc=1,
                device_id=(nbr,), device_id_type=pl.DeviceIdType.MESH,
            )
        pl.semaphore_wait(barrier_sem, 2)

        q = jnp.dot(x_ref[...], wq_ref[...], preferred_element_type=jnp.float32)

        qi = lax.broadcasted_iota(jnp.int32, (SQ, SKV), 0)
        ki = lax.broadcasted_iota(jnp.int32, (SQ, SKV), 1)
        mask = (jnp.abs(qi - ki) <= 128) | (ki < 32) | (qi < 32)

        for h in range(HQ_LOCAL):
            q_h = q[:, h * DH:(h + 1) * DH]
            k_h = k_ref[:, h * DH:(h + 1) * DH]
            s_h = lax.dot_general(
                q_h, k_h, (((1,), (1,)), ((), ())),
                preferred_element_type=jnp.float32,
            ) * SCALE
            s_h = jnp.where(mask, s_h, -1e9)
            m = jnp.max(s_h, axis=1, keepdims=True)
            w = jnp.exp(s_h - m)
            w = w / jnp.sum(w, axis=1, keepdims=True)
            ctx_ref[:, h * DH:(h + 1) * DH] = jnp.dot(
                w, v_ref[:, h * DH:(h + 1) * DH],
                preferred_element_type=jnp.float32,
            )

        acc_ref[...] = jnp.dot(
            ctx_ref[...], wo_ref[...], preferred_element_type=jnp.float32
        )

        for s in range(N_DEV - 1):
            c_send = lax.rem(my - s + N_DEV, N_DEV)
            rdma = pltpu.make_async_remote_copy(
                src_ref=acc_ref.at[pl.ds(c_send * CHUNK, CHUNK), :],
                dst_ref=comm_ref.at[s],
                send_sem=rs_send.at[s],
                recv_sem=rs_recv.at[s],
                device_id=(right,),
                device_id_type=pl.DeviceIdType.MESH,
            )
            rdma.start()
            rdma.wait()
            c_recv = lax.rem(my - s - 1 + N_DEV, N_DEV)
            sl = pl.ds(c_recv * CHUNK, CHUNK)
            acc_ref[sl, :] = acc_ref[sl, :] + comm_ref[s]

        c_own = lax.rem(my + 1, N_DEV)
        own_sl = pl.ds(c_own * CHUNK, CHUNK)
        out_ref[own_sl, :] = acc_ref[own_sl, :]

        for s in range(N_DEV - 1):
            c_send = lax.rem(my + 1 - s + N_DEV, N_DEV)
            sl = pl.ds(c_send * CHUNK, CHUNK)
            rdma = pltpu.make_async_remote_copy(
                src_ref=out_ref.at[sl, :],
                dst_ref=out_ref.at[sl, :],
                send_sem=ag_send.at[s],
                recv_sem=ag_recv.at[s],
                device_id=(right,),
                device_id_type=pl.DeviceIdType.MESH,
            )
            rdma.start()
            rdma.wait()

    out = pl.pallas_call(
        body,
        out_shape=jax.ShapeDtypeStruct((SQ, D_MODEL), jnp.float32),
        in_specs=[pl.BlockSpec(memory_space=pltpu.VMEM)] * 5,
        out_specs=pl.BlockSpec(memory_space=pltpu.VMEM),
        scratch_shapes=[
            pltpu.VMEM((SQ, D_MODEL), jnp.float32),
            pltpu.VMEM((SQ, HQ_LOCAL * DH), jnp.float32),
            pltpu.VMEM((N_DEV - 1, CHUNK, D_MODEL), jnp.float32),
            pltpu.SemaphoreType.DMA((N_DEV - 1,)),
            pltpu.SemaphoreType.DMA((N_DEV - 1,)),
            pltpu.SemaphoreType.DMA((N_DEV - 1,)),
            pltpu.SemaphoreType.DMA((N_DEV - 1,)),
        ],
        compiler_params=pltpu.CompilerParams(collective_id=0),
    )(x2, wq_sl, k2, v2, wo_sl)

    return out[None]


# device time: 141071 ns/iter; 1.2631x vs baseline; 1.2631x over previous
import jax
import jax.numpy as jnp
from jax import lax
from jax.experimental import pallas as pl
from jax.experimental.pallas import tpu as pltpu

N_DEV = 16
SQ = 1024
SKV = 1024
HQ_LOCAL = 8
DH = 128
D_MODEL = 1024
SCALE = 0.08838834764831843


def kernel(x, Wq, K_ext, V_ext, Wo):
    i = lax.axis_index("i")

    x2 = x[0]
    wq_sl = lax.dynamic_slice(Wq, (0, i * D_MODEL), (D_MODEL, D_MODEL))
    wo_sl = lax.dynamic_slice(Wo, (i * D_MODEL, 0), (D_MODEL, D_MODEL))
    k2 = K_ext[0].reshape(SKV, HQ_LOCAL * DH)
    v2 = V_ext[0].reshape(SKV, HQ_LOCAL * DH)

    def body(x_ref, wq_ref, k_ref, v_ref, wo_ref, out_ref,
             acc_ref, ctx_ref, comm_ref, *sems):
        rs_sems = [(sems[2 * s], sems[2 * s + 1]) for s in range(4)]
        ag_sems = [(sems[8 + 2 * s], sems[9 + 2 * s]) for s in range(4)]

        my = lax.axis_index("i")
        p = my % 4
        z = my // 4
        bx = (p ^ (p >> 1)) & 1
        by = (p >> 1) & 1
        bz0 = z & 1
        bz1 = (z >> 1) & 1

        partner_x = my ^ 1
        partner_y = my + 3 - 2 * p
        partner_z0 = my ^ 4
        partner_z1 = my ^ 8
        partners = [partner_x, partner_y, partner_z0, partner_z1]

        barrier_sem = pltpu.get_barrier_semaphore()
        for nbr in partners:
            pl.semaphore_signal(
                barrier_sem, inc=1,
                device_id=(nbr,), device_id_type=pl.DeviceIdType.MESH,
            )
        pl.semaphore_wait(barrier_sem, len(partners))

        q = jnp.dot(x_ref[...], wq_ref[...], preferred_element_type=jnp.float32)

        qi = lax.broadcasted_iota(jnp.int32, (SQ, SKV), 0)
        ki = lax.broadcasted_iota(jnp.int32, (SQ, SKV), 1)
        mask = (jnp.abs(qi - ki) <= 128) | (ki < 32) | (qi < 32)

        for h in range(HQ_LOCAL):
            q_h = q[:, h * DH:(h + 1) * DH]
            k_h = k_ref[:, h * DH:(h + 1) * DH]
            s_h = lax.dot_general(
                q_h, k_h, (((1,), (1,)), ((), ())),
                preferred_element_type=jnp.float32,
            ) * SCALE
            s_h = jnp.where(mask, s_h, -1e9)
            m = jnp.max(s_h, axis=1, keepdims=True)
            w = jnp.exp(s_h - m)
            w = w / jnp.sum(w, axis=1, keepdims=True)
            ctx_ref[:, h * DH:(h + 1) * DH] = jnp.dot(
                w, v_ref[:, h * DH:(h + 1) * DH],
                preferred_element_type=jnp.float32,
            )

        acc_ref[...] = jnp.dot(
            ctx_ref[...], wo_ref[...], preferred_element_type=jnp.float32
        )

        base1 = bx * 512
        base2 = base1 + by * 256
        base3 = base2 + bz0 * 128
        own = base3 + bz1 * 64
        rs_stages = [
            (partner_x, (1 - bx) * 512, base1, 512, 0),
            (partner_y, base1 + (1 - by) * 256, base2, 256, 512),
            (partner_z0, base2 + (1 - bz0) * 128, base3, 128, 768),
            (partner_z1, base3 + (1 - bz1) * 64, own, 64, 896),
        ]
        for s, (partner, send_off, keep_off, size, comm_off) in enumerate(rs_stages):
            rdma = pltpu.make_async_remote_copy(
                src_ref=acc_ref.at[pl.ds(send_off, size), :],
                dst_ref=comm_ref.at[pl.ds(comm_off, size), :],
                send_sem=rs_sems[s][0],
                recv_sem=rs_sems[s][1],
                device_id=(partner,),
                device_id_type=pl.DeviceIdType.MESH,
            )
            rdma.start()
            rdma.wait()
            ksl = pl.ds(keep_off, size)
            csl = pl.ds(comm_off, size)
            acc_ref[ksl, :] = acc_ref[ksl, :] + comm_ref[csl, :]

        own_sl = pl.ds(own, 64)
        out_ref[own_sl, :] = acc_ref[own_sl, :]
        ag_stages = [
            (partner_z1, own, 64),
            (partner_z0, base3, 128),
            (partner_y, base2, 256),
            (partner_x, base1, 512),
        ]
        for s, (partner, off, size) in enumerate(ag_stages):
            sl = pl.ds(off, size)
            rdma = pltpu.make_async_remote_copy(
                src_ref=out_ref.at[sl, :],
                dst_ref=out_ref.at[sl, :],
                send_sem=ag_sems[s][0],
                recv_sem=ag_sems[s][1],
                device_id=(partner,),
                device_id_type=pl.DeviceIdType.MESH,
            )
            rdma.start()
            rdma.wait()

    out = pl.pallas_call(
        body,
        out_shape=jax.ShapeDtypeStruct((SQ, D_MODEL), jnp.float32),
        in_specs=[pl.BlockSpec(memory_space=pltpu.VMEM)] * 5,
        out_specs=pl.BlockSpec(memory_space=pltpu.VMEM),
        scratch_shapes=[
            pltpu.VMEM((SQ, D_MODEL), jnp.float32),
            pltpu.VMEM((SQ, HQ_LOCAL * DH), jnp.float32),
            pltpu.VMEM((960, D_MODEL), jnp.float32),
        ] + [pltpu.SemaphoreType.DMA(())] * 16,
        compiler_params=pltpu.CompilerParams(collective_id=0),
    )(x2, wq_sl, k2, v2, wo_sl)

    return out[None]


# device time: 97824 ns/iter; 1.8215x vs baseline; 1.4421x over previous
import jax
import jax.numpy as jnp
from jax import lax
from jax.experimental import pallas as pl
from jax.experimental.pallas import tpu as pltpu

N_DEV = 16
SQ = 1024
SKV = 1024
HQ_LOCAL = 8
DH = 128
D_MODEL = 1024
SCALE = 0.08838834764831843


def kernel(x, Wq, K_ext, V_ext, Wo):
    i = lax.axis_index("i")

    x2 = x[0]
    wq_sl = lax.dynamic_slice(Wq, (0, i * D_MODEL), (D_MODEL, D_MODEL))
    wo_sl = lax.dynamic_slice(Wo, (i * D_MODEL, 0), (D_MODEL, D_MODEL))
    k2 = K_ext[0].reshape(SKV, HQ_LOCAL * DH)
    v2 = V_ext[0].reshape(SKV, HQ_LOCAL * DH)

    def body(x_ref, wq_ref, k_ref, v_ref, wo_ref, out_ref,
             acc_ref, ctx_ref, comm_ref, sbuf_ref, ob_ref, *sems):
        rs_sems = [(sems[2 * s], sems[2 * s + 1]) for s in range(4)]
        ag_sems = [(sems[8 + 2 * s], sems[9 + 2 * s]) for s in range(4)]

        my = lax.axis_index("i")
        p = my % 4
        z = my // 4
        bx = (p ^ (p >> 1)) & 1
        by = (p >> 1) & 1
        bz0 = z & 1
        bz1 = (z >> 1) & 1

        partner_x = my ^ 1
        partner_y = my + 3 - 2 * p
        partner_z0 = my ^ 4
        partner_z1 = my ^ 8
        partners = [partner_x, partner_y, partner_z0, partner_z1]

        barrier_sem = pltpu.get_barrier_semaphore()
        for nbr in partners:
            pl.semaphore_signal(
                barrier_sem, inc=1,
                device_id=(nbr,), device_id_type=pl.DeviceIdType.MESH,
            )
        pl.semaphore_wait(barrier_sem, len(partners))

        q = jnp.dot(x_ref[...], wq_ref[...], preferred_element_type=jnp.float32)

        qi = lax.broadcasted_iota(jnp.int32, (SQ, SKV), 0)
        ki = lax.broadcasted_iota(jnp.int32, (SQ, SKV), 1)
        mask = (jnp.abs(qi - ki) <= 128) | (ki < 32) | (qi < 32)

        for h in range(HQ_LOCAL):
            q_h = q[:, h * DH:(h + 1) * DH]
            k_h = k_ref[:, h * DH:(h + 1) * DH]
            s_h = lax.dot_general(
                q_h, k_h, (((1,), (1,)), ((), ())),
                preferred_element_type=jnp.float32,
            ) * SCALE
            s_h = jnp.where(mask, s_h, -1e9)
            m = jnp.max(s_h, axis=1, keepdims=True)
            w = jnp.exp(s_h - m)
            w = w / jnp.sum(w, axis=1, keepdims=True)
            ctx_ref[:, h * DH:(h + 1) * DH] = jnp.dot(
                w, v_ref[:, h * DH:(h + 1) * DH],
                preferred_element_type=jnp.float32,
            )

        acc_ref[...] = jnp.dot(
            ctx_ref[...], wo_ref[...], preferred_element_type=jnp.float32
        )

        base1 = bx * 512
        base2 = base1 + by * 256
        base3 = base2 + bz0 * 128
        own = base3 + bz1 * 64
        rs_stages = [
            (partner_x, (1 - bx) * 512, base1, 512, 0),
            (partner_y, base1 + (1 - by) * 256, base2, 256, 512),
            (partner_z0, base2 + (1 - bz0) * 128, base3, 128, 768),
            (partner_z1, base3 + (1 - bz1) * 64, own, 64, 896),
        ]
        for s, (partner, send_off, keep_off, size, comm_off) in enumerate(rs_stages):
            ssl = pl.ds(0, size)
            sbuf_ref[ssl, :] = acc_ref[pl.ds(send_off, size), :].astype(
                jnp.bfloat16
            )
            rdma = pltpu.make_async_remote_copy(
                src_ref=sbuf_ref.at[ssl, :],
                dst_ref=comm_ref.at[pl.ds(comm_off, size), :],
                send_sem=rs_sems[s][0],
                recv_sem=rs_sems[s][1],
                device_id=(partner,),
                device_id_type=pl.DeviceIdType.MESH,
            )
            rdma.start()
            rdma.wait()
            ksl = pl.ds(keep_off, size)
            csl = pl.ds(comm_off, size)
            acc_ref[ksl, :] = acc_ref[ksl, :] + comm_ref[csl, :].astype(
                jnp.float32
            )

        own_sl = pl.ds(own, 64)
        ob_ref[own_sl, :] = acc_ref[own_sl, :].astype(jnp.bfloat16)
        ag_stages = [
            (partner_z1, own, 64),
            (partner_z0, base3, 128),
            (partner_y, base2, 256),
            (partner_x, base1, 512),
        ]
        for s, (partner, off, size) in enumerate(ag_stages):
            sl = pl.ds(off, size)
            rdma = pltpu.make_async_remote_copy(
                src_ref=ob_ref.at[sl, :],
                dst_ref=ob_ref.at[sl, :],
                send_sem=ag_sems[s][0],
                recv_sem=ag_sems[s][1],
                device_id=(partner,),
                device_id_type=pl.DeviceIdType.MESH,
            )
            rdma.start()
            rdma.wait()

        out_ref[...] = ob_ref[...].astype(jnp.float32)

    out = pl.pallas_call(
        body,
        out_shape=jax.ShapeDtypeStruct((SQ, D_MODEL), jnp.float32),
        in_specs=[pl.BlockSpec(memory_space=pltpu.VMEM)] * 5,
        out_specs=pl.BlockSpec(memory_space=pltpu.VMEM),
        scratch_shapes=[
            pltpu.VMEM((SQ, D_MODEL), jnp.float32),
            pltpu.VMEM((SQ, HQ_LOCAL * DH), jnp.float32),
            pltpu.VMEM((960, D_MODEL), jnp.bfloat16),
            pltpu.VMEM((512, D_MODEL), jnp.bfloat16),
            pltpu.VMEM((SQ, D_MODEL), jnp.bfloat16),
        ] + [pltpu.SemaphoreType.DMA(())] * 16,
        compiler_params=pltpu.CompilerParams(collective_id=0),
    )(x2, wq_sl, k2, v2, wo_sl)

    return out[None]


# device time: 94470 ns/iter; 1.8862x vs baseline; 1.0355x over previous
import jax
import jax.numpy as jnp
from jax import lax
from jax.experimental import pallas as pl
from jax.experimental.pallas import tpu as pltpu

N_DEV = 16
SQ = 1024
SKV = 1024
HQ_LOCAL = 8
DH = 128
D_MODEL = 1024
SCALE = 0.08838834764831843


def kernel(x, Wq, K_ext, V_ext, Wo):
    i = lax.axis_index("i")

    x2 = x[0].astype(jnp.bfloat16)
    wq_sl = lax.dynamic_slice(Wq, (0, i * D_MODEL), (D_MODEL, D_MODEL)).astype(
        jnp.bfloat16
    )
    wo_sl = lax.dynamic_slice(Wo, (i * D_MODEL, 0), (D_MODEL, D_MODEL)).astype(
        jnp.bfloat16
    )
    k2 = K_ext[0].reshape(SKV, HQ_LOCAL * DH).astype(jnp.bfloat16)
    v2 = V_ext[0].reshape(SKV, HQ_LOCAL * DH).astype(jnp.bfloat16)

    def body(x_ref, wq_ref, k_ref, v_ref, wo_ref, out_ref,
             acc_ref, ctx_ref, comm_ref, sbuf_ref, ob_ref, *sems):
        rs_sems = [(sems[2 * s], sems[2 * s + 1]) for s in range(4)]
        ag_sems = [(sems[8 + 2 * s], sems[9 + 2 * s]) for s in range(4)]

        my = lax.axis_index("i")
        p = my % 4
        z = my // 4
        bx = (p ^ (p >> 1)) & 1
        by = (p >> 1) & 1
        bz0 = z & 1
        bz1 = (z >> 1) & 1

        partner_x = my ^ 1
        partner_y = my + 3 - 2 * p
        partner_z0 = my ^ 4
        partner_z1 = my ^ 8
        partners = [partner_x, partner_y, partner_z0, partner_z1]

        barrier_sem = pltpu.get_barrier_semaphore()
        for nbr in partners:
            pl.semaphore_signal(
                barrier_sem, inc=1,
                device_id=(nbr,), device_id_type=pl.DeviceIdType.MESH,
            )
        pl.semaphore_wait(barrier_sem, len(partners))

        q = jnp.dot(
            x_ref[...], wq_ref[...], preferred_element_type=jnp.float32
        ).astype(jnp.bfloat16)

        qi = lax.broadcasted_iota(jnp.int32, (SQ, SKV), 0)
        ki = lax.broadcasted_iota(jnp.int32, (SQ, SKV), 1)
        mask = (jnp.abs(qi - ki) <= 128) | (ki < 32) | (qi < 32)

        for h in range(HQ_LOCAL):
            q_h = q[:, h * DH:(h + 1) * DH]
            k_h = k_ref[:, h * DH:(h + 1) * DH]
            s_h = lax.dot_general(
                q_h, k_h, (((1,), (1,)), ((), ())),
                preferred_element_type=jnp.float32,
            ) * SCALE
            s_h = jnp.where(mask, s_h, -1e9)
            m = jnp.max(s_h, axis=1, keepdims=True)
            w = jnp.exp(s_h - m)
            w = (w / jnp.sum(w, axis=1, keepdims=True)).astype(jnp.bfloat16)
            ctx_ref[:, h * DH:(h + 1) * DH] = jnp.dot(
                w, v_ref[:, h * DH:(h + 1) * DH],
                preferred_element_type=jnp.float32,
            ).astype(jnp.bfloat16)

        acc_ref[...] = jnp.dot(
            ctx_ref[...], wo_ref[...], preferred_element_type=jnp.float32
        )

        base1 = bx * 512
        base2 = base1 + by * 256
        base3 = base2 + bz0 * 128
        own = base3 + bz1 * 64
        rs_stages = [
            (partner_x, (1 - bx) * 512, base1, 512, 0),
            (partner_y, base1 + (1 - by) * 256, base2, 256, 512),
            (partner_z0, base2 + (1 - bz0) * 128, base3, 128, 768),
            (partner_z1, base3 + (1 - bz1) * 64, own, 64, 896),
        ]
        for s, (partner, send_off, keep_off, size, comm_off) in enumerate(rs_stages):
            ssl = pl.ds(0, size)
            sbuf_ref[ssl, :] = acc_ref[pl.ds(send_off, size), :].astype(
                jnp.bfloat16
            )
            rdma = pltpu.make_async_remote_copy(
                src_ref=sbuf_ref.at[ssl, :],
                dst_ref=comm_ref.at[pl.ds(comm_off, size), :],
                send_sem=rs_sems[s][0],
                recv_sem=rs_sems[s][1],
                device_id=(partner,),
                device_id_type=pl.DeviceIdType.MESH,
            )
            rdma.start()
            rdma.wait()
            ksl = pl.ds(keep_off, size)
            csl = pl.ds(comm_off, size)
            acc_ref[ksl, :] = acc_ref[ksl, :] + comm_ref[csl, :].astype(
                jnp.float32
            )

        own_sl = pl.ds(own, 64)
        ob_ref[own_sl, :] = acc_ref[own_sl, :].astype(jnp.bfloat16)
        ag_stages = [
            (partner_z1, own, 64),
            (partner_z0, base3, 128),
            (partner_y, base2, 256),
            (partner_x, base1, 512),
        ]
        for s, (partner, off, size) in enumerate(ag_stages):
            sl = pl.ds(off, size)
            rdma = pltpu.make_async_remote_copy(
                src_ref=ob_ref.at[sl, :],
                dst_ref=ob_ref.at[sl, :],
                send_sem=ag_sems[s][0],
                recv_sem=ag_sems[s][1],
                device_id=(partner,),
                device_id_type=pl.DeviceIdType.MESH,
            )
            rdma.start()
            rdma.wait()

        out_ref[...] = ob_ref[...].astype(jnp.float32)

    out = pl.pallas_call(
        body,
        out_shape=jax.ShapeDtypeStruct((SQ, D_MODEL), jnp.float32),
        in_specs=[pl.BlockSpec(memory_space=pltpu.VMEM)] * 5,
        out_specs=pl.BlockSpec(memory_space=pltpu.VMEM),
        scratch_shapes=[
            pltpu.VMEM((SQ, D_MODEL), jnp.float32),
            pltpu.VMEM((SQ, HQ_LOCAL * DH), jnp.bfloat16),
            pltpu.VMEM((960, D_MODEL), jnp.bfloat16),
            pltpu.VMEM((512, D_MODEL), jnp.bfloat16),
            pltpu.VMEM((SQ, D_MODEL), jnp.bfloat16),
        ] + [pltpu.SemaphoreType.DMA(())] * 16,
        compiler_params=pltpu.CompilerParams(collective_id=0),
    )(x2, wq_sl, k2, v2, wo_sl)

    return out[None]


# device time: 91047 ns/iter; 1.9571x vs baseline; 1.0376x over previous
import jax
import jax.numpy as jnp
from jax import lax
from jax.experimental import pallas as pl
from jax.experimental.pallas import tpu as pltpu

N_DEV = 16
SQ = 1024
SKV = 1024
HQ_LOCAL = 8
DH = 128
D_MODEL = 1024
SCALE = 0.08838834764831843


def kernel(x, Wq, K_ext, V_ext, Wo):
    i = lax.axis_index("i")

    x2 = x[0].astype(jnp.bfloat16)
    wq_sl = lax.dynamic_slice(Wq, (0, i * D_MODEL), (D_MODEL, D_MODEL)).astype(
        jnp.bfloat16
    )
    wo_sl = lax.dynamic_slice(Wo, (i * D_MODEL, 0), (D_MODEL, D_MODEL)).astype(
        jnp.bfloat16
    )
    k2 = K_ext[0].reshape(SKV, HQ_LOCAL * DH).astype(jnp.bfloat16)
    v2 = V_ext[0].reshape(SKV, HQ_LOCAL * DH).astype(jnp.bfloat16)

    def body(x_ref, wq_ref, k_ref, v_ref, wo_ref, out_ref,
             acc_ref, ctx_ref, comm_ref, sbuf_ref, ob_ref, *sems):
        rs_sems = [(sems[2 * s], sems[2 * s + 1]) for s in range(4)]
        ag_sems = [(sems[8 + 2 * s], sems[9 + 2 * s]) for s in range(4)]

        my = lax.axis_index("i")
        p = my % 4
        z = my // 4
        bx = (p ^ (p >> 1)) & 1
        by = (p >> 1) & 1
        bz0 = z & 1
        bz1 = (z >> 1) & 1

        partner_x = my ^ 1
        partner_y = my + 3 - 2 * p
        partner_z0 = my ^ 4
        partner_z1 = my ^ 8
        partners = [partner_x, partner_y, partner_z0, partner_z1]

        barrier_sem = pltpu.get_barrier_semaphore()
        for nbr in partners:
            pl.semaphore_signal(
                barrier_sem, inc=1,
                device_id=(nbr,), device_id_type=pl.DeviceIdType.MESH,
            )
        pl.semaphore_wait(barrier_sem, len(partners))

        q = jnp.dot(
            x_ref[...], wq_ref[...], preferred_element_type=jnp.float32
        ).astype(jnp.bfloat16)

        qi = lax.broadcasted_iota(jnp.int32, (SQ, SKV), 0)
        ki = lax.broadcasted_iota(jnp.int32, (SQ, SKV), 1)
        mask = (jnp.abs(qi - ki) <= 128) | (ki < 32) | (qi < 32)

        for h in range(HQ_LOCAL):
            q_h = q[:, h * DH:(h + 1) * DH]
            k_h = k_ref[:, h * DH:(h + 1) * DH]
            s_h = lax.dot_general(
                q_h, k_h, (((1,), (1,)), ((), ())),
                preferred_element_type=jnp.float32,
            ) * SCALE
            s_h = jnp.where(mask, s_h, -1e9)
            e = jnp.exp(s_h)
            denom = jnp.sum(e, axis=1, keepdims=True)
            ctx_h = jnp.dot(
                e.astype(jnp.bfloat16), v_ref[:, h * DH:(h + 1) * DH],
                preferred_element_type=jnp.float32,
            )
            ctx_ref[:, h * DH:(h + 1) * DH] = (ctx_h / denom).astype(
                jnp.bfloat16
            )

        acc_ref[...] = jnp.dot(
            ctx_ref[...], wo_ref[...], preferred_element_type=jnp.float32
        )

        base1 = bx * 512
        base2 = base1 + by * 256
        base3 = base2 + bz0 * 128
        own = base3 + bz1 * 64
        rs_stages = [
            (partner_x, (1 - bx) * 512, base1, 512, 0),
            (partner_y, base1 + (1 - by) * 256, base2, 256, 512),
            (partner_z0, base2 + (1 - bz0) * 128, base3, 128, 768),
            (partner_z1, base3 + (1 - bz1) * 64, own, 64, 896),
        ]
        for s, (partner, send_off, keep_off, size, comm_off) in enumerate(rs_stages):
            ssl = pl.ds(0, size)
            sbuf_ref[ssl, :] = acc_ref[pl.ds(send_off, size), :].astype(
                jnp.bfloat16
            )
            rdma = pltpu.make_async_remote_copy(
                src_ref=sbuf_ref.at[ssl, :],
                dst_ref=comm_ref.at[pl.ds(comm_off, size), :],
                send_sem=rs_sems[s][0],
                recv_sem=rs_sems[s][1],
                device_id=(partner,),
                device_id_type=pl.DeviceIdType.MESH,
            )
            rdma.start()
            rdma.wait()
            ksl = pl.ds(keep_off, size)
            csl = pl.ds(comm_off, size)
            acc_ref[ksl, :] = acc_ref[ksl, :] + comm_ref[csl, :].astype(
                jnp.float32
            )

        own_sl = pl.ds(own, 64)
        ob_ref[own_sl, :] = acc_ref[own_sl, :].astype(jnp.bfloat16)
        ag_stages = [
            (partner_z1, own, 64),
            (partner_z0, base3, 128),
            (partner_y, base2, 256),
            (partner_x, base1, 512),
        ]
        for s, (partner, off, size) in enumerate(ag_stages):
            sl = pl.ds(off, size)
            rdma = pltpu.make_async_remote_copy(
                src_ref=ob_ref.at[sl, :],
                dst_ref=ob_ref.at[sl, :],
                send_sem=ag_sems[s][0],
                recv_sem=ag_sems[s][1],
                device_id=(partner,),
                device_id_type=pl.DeviceIdType.MESH,
            )
            rdma.start()
            rdma.wait()

        out_ref[...] = ob_ref[...].astype(jnp.float32)

    out = pl.pallas_call(
        body,
        out_shape=jax.ShapeDtypeStruct((SQ, D_MODEL), jnp.float32),
        in_specs=[pl.BlockSpec(memory_space=pltpu.VMEM)] * 5,
        out_specs=pl.BlockSpec(memory_space=pltpu.VMEM),
        scratch_shapes=[
            pltpu.VMEM((SQ, D_MODEL), jnp.float32),
            pltpu.VMEM((SQ, HQ_LOCAL * DH), jnp.bfloat16),
            pltpu.VMEM((960, D_MODEL), jnp.bfloat16),
            pltpu.VMEM((512, D_MODEL), jnp.bfloat16),
            pltpu.VMEM((SQ, D_MODEL), jnp.bfloat16),
        ] + [pltpu.SemaphoreType.DMA(())] * 16,
        compiler_params=pltpu.CompilerParams(collective_id=0),
    )(x2, wq_sl, k2, v2, wo_sl)

    return out[None]


# device time: 84516 ns/iter; 2.1083x vs baseline; 1.0773x over previous
import jax
import jax.numpy as jnp
from jax import lax
from jax.experimental import pallas as pl
from jax.experimental.pallas import tpu as pltpu

N_DEV = 16
SQ = 1024
SKV = 1024
HQ_LOCAL = 8
DH = 128
D_MODEL = 1024
SCALE = 0.08838834764831843


def kernel(x, Wq, K_ext, V_ext, Wo):
    i = lax.axis_index("i")

    x2 = x[0].astype(jnp.bfloat16)
    wq_sl = lax.dynamic_slice(Wq, (0, i * D_MODEL), (D_MODEL, D_MODEL)).astype(
        jnp.bfloat16
    )
    wo_sl = lax.dynamic_slice(Wo, (i * D_MODEL, 0), (D_MODEL, D_MODEL)).astype(
        jnp.bfloat16
    )
    k2 = K_ext[0].reshape(SKV, HQ_LOCAL * DH).astype(jnp.bfloat16)
    v2 = V_ext[0].reshape(SKV, HQ_LOCAL * DH).astype(jnp.bfloat16)

    def body(x_ref, wq_ref, k_ref, v_ref, wo_ref, out_ref,
             acc_ref, ctx_ref, comm_ref, sbuf_ref, ob_ref, *sems):
        rs_sems = [(sems[2 * s], sems[2 * s + 1]) for s in range(4)]
        ag_sems = [(sems[8 + 2 * s], sems[9 + 2 * s]) for s in range(4)]

        my = lax.axis_index("i")
        p = my % 4
        z = my // 4
        bx = (p ^ (p >> 1)) & 1
        by = (p >> 1) & 1
        bz0 = z & 1
        bz1 = (z >> 1) & 1

        partner_x = my ^ 1
        partner_y = my + 3 - 2 * p
        partner_z0 = my ^ 4
        partner_z1 = my ^ 8
        partners = [partner_x, partner_y, partner_z0, partner_z1]

        barrier_sem = pltpu.get_barrier_semaphore()
        for nbr in partners:
            pl.semaphore_signal(
                barrier_sem, inc=1,
                device_id=(nbr,), device_id_type=pl.DeviceIdType.MESH,
            )
        pl.semaphore_wait(barrier_sem, len(partners))

        def compute_half(off):
            rsl = pl.ds(off, 512)
            qh = jnp.dot(
                x_ref[rsl, :], wq_ref[...], preferred_element_type=jnp.float32
            ).astype(jnp.bfloat16)
            qi = lax.broadcasted_iota(jnp.int32, (512, SKV), 0) + off
            ki = lax.broadcasted_iota(jnp.int32, (512, SKV), 1)
            mask = (jnp.abs(qi - ki) <= 128) | (ki < 32) | (qi < 32)
            for h in range(HQ_LOCAL):
                hsl = pl.ds(h * DH, DH)
                s_h = lax.dot_general(
                    qh[:, h * DH:(h + 1) * DH], k_ref[:, hsl],
                    (((1,), (1,)), ((), ())),
                    preferred_element_type=jnp.float32,
                ) * SCALE
                s_h = jnp.where(mask, s_h, -1e9)
                e = jnp.exp(s_h)
                denom = jnp.sum(e, axis=1, keepdims=True)
                ctx_h = jnp.dot(
                    e.astype(jnp.bfloat16), v_ref[:, hsl],
                    preferred_element_type=jnp.float32,
                )
                ctx_ref[rsl, hsl] = (ctx_h / denom).astype(jnp.bfloat16)
            acc_ref[rsl, :] = jnp.dot(
                ctx_ref[rsl, :], wo_ref[...],
                preferred_element_type=jnp.float32,
            )

        base1 = bx * 512
        base2 = base1 + by * 256
        base3 = base2 + bz0 * 128
        own = base3 + bz1 * 64

        send1 = (1 - bx) * 512
        compute_half(send1)
        sbuf_ref[pl.ds(0, 512), :] = acc_ref[pl.ds(send1, 512), :].astype(
            jnp.bfloat16
        )
        rdma1 = pltpu.make_async_remote_copy(
            src_ref=sbuf_ref.at[pl.ds(0, 512), :],
            dst_ref=comm_ref.at[pl.ds(0, 512), :],
            send_sem=rs_sems[0][0],
            recv_sem=rs_sems[0][1],
            device_id=(partner_x,),
            device_id_type=pl.DeviceIdType.MESH,
        )
        rdma1.start()
        compute_half(base1)
        rdma1.wait()
        k1 = pl.ds(base1, 512)
        acc_ref[k1, :] = acc_ref[k1, :] + comm_ref[pl.ds(0, 512), :].astype(
            jnp.float32
        )

        rs_stages = [
            (partner_y, base1 + (1 - by) * 256, base2, 256, 512),
            (partner_z0, base2 + (1 - bz0) * 128, base3, 128, 768),
            (partner_z1, base3 + (1 - bz1) * 64, own, 64, 896),
        ]
        for s, (partner, send_off, keep_off, size, comm_off) in enumerate(
            rs_stages, start=1
        ):
            ssl = pl.ds(0, size)
            sbuf_ref[ssl, :] = acc_ref[pl.ds(send_off, size), :].astype(
                jnp.bfloat16
            )
            rdma = pltpu.make_async_remote_copy(
                src_ref=sbuf_ref.at[ssl, :],
                dst_ref=comm_ref.at[pl.ds(comm_off, size), :],
                send_sem=rs_sems[s][0],
                recv_sem=rs_sems[s][1],
                device_id=(partner,),
                device_id_type=pl.DeviceIdType.MESH,
            )
            rdma.start()
            rdma.wait()
            ksl = pl.ds(keep_off, size)
            csl = pl.ds(comm_off, size)
            acc_ref[ksl, :] = acc_ref[ksl, :] + comm_ref[csl, :].astype(
                jnp.float32
            )

        own_sl = pl.ds(own, 64)
        ob_ref[own_sl, :] = acc_ref[own_sl, :].astype(jnp.bfloat16)
        ag_stages = [
            (partner_z1, own, 64),
            (partner_z0, base3, 128),
            (partner_y, base2, 256),
            (partner_x, base1, 512),
        ]
        for s, (partner, off, size) in enumerate(ag_stages):
            sl = pl.ds(off, size)
            rdma = pltpu.make_async_remote_copy(
                src_ref=ob_ref.at[sl, :],
                dst_ref=ob_ref.at[sl, :],
                send_sem=ag_sems[s][0],
                recv_sem=ag_sems[s][1],
                device_id=(partner,),
                device_id_type=pl.DeviceIdType.MESH,
            )
            rdma.start()
            rdma.wait()

        out_ref[...] = ob_ref[...].astype(jnp.float32)

    out = pl.pallas_call(
        body,
        out_shape=jax.ShapeDtypeStruct((SQ, D_MODEL), jnp.float32),
        in_specs=[pl.BlockSpec(memory_space=pltpu.VMEM)] * 5,
        out_specs=pl.BlockSpec(memory_space=pltpu.VMEM),
        scratch_shapes=[
            pltpu.VMEM((SQ, D_MODEL), jnp.float32),
            pltpu.VMEM((SQ, HQ_LOCAL * DH), jnp.bfloat16),
            pltpu.VMEM((960, D_MODEL), jnp.bfloat16),
            pltpu.VMEM((512, D_MODEL), jnp.bfloat16),
            pltpu.VMEM((SQ, D_MODEL), jnp.bfloat16),
        ] + [pltpu.SemaphoreType.DMA(())] * 16,
        compiler_params=pltpu.CompilerParams(collective_id=0),
    )(x2, wq_sl, k2, v2, wo_sl)

    return out[None]
